# baseline (speedup 1.0000x reference)
import functools

import jax
import jax.numpy as jnp
from jax import lax
from jax.experimental import pallas as pl
from jax.experimental.pallas import tpu as pltpu
from jax.experimental.pallas import tpu_sc as plsc

B = 128
N = 32768
N2 = N // 2
K = 30
UNROLL = 4
NGROUP2 = N2 // (16 * UNROLL)
CAP = 4096
EPS = 1e-07
LN2 = 0.6931471805599453
NEG_BIG = -3.4e38


def _sort_asc(v):
    return plsc.sort_key_val(v, v)[0]


def _sort_desc(v):
    return plsc.sort_key_val(v, v, descending=True)[0]


def _lane(vec, i):
    li = lax.iota(jnp.int32, 16)
    return jnp.max(jnp.where(li == i, vec, NEG_BIG))


def _merge_chunk(v, state):
    H, L, th = state
    hit = jnp.any(v > th)

    def merge(c):
        H, L, _ = c
        vs = _sort_asc(v)
        up = jnp.maximum(vs, L)
        upd = _sort_desc(up)
        nH = _sort_asc(jnp.maximum(H, upd))
        nL = _sort_desc(jnp.minimum(H, upd))
        return (nH, nL, jnp.min(nL))

    return lax.cond(hit, merge, lambda c: c, (H, L, th))


def _hl_init(c0, c1):
    h0 = _sort_asc(c0)
    l0 = _sort_desc(c1)
    H = _sort_asc(jnp.maximum(h0, l0))
    L = _sort_desc(jnp.minimum(h0, l0))
    return (H, L, jnp.min(L))


def _hl_fin(st):
    H, L, _ = st
    tv = _lane(L, K - 1 - 16)
    cgt = (jnp.sum((H > tv).astype(jnp.int32))
           + jnp.sum((L > tv).astype(jnp.int32)))
    return tv, K - cgt


def _neg_log_sigmoid(x):
    s = 1.0 / (1.0 + jnp.exp(-x))
    y = s + jnp.float32(EPS)
    bits = plsc.bitcast(y, jnp.int32)
    e = (bits >> 23) - 127
    m = plsc.bitcast((bits & 0x7FFFFF) | 0x3F800000, jnp.float32)
    z = (m - 1.0) / (m + 1.0)
    z2 = z * z
    p = 1.0 + z2 * (jnp.float32(1 / 3) + z2 * (jnp.float32(1 / 5)
          + z2 * (jnp.float32(1 / 7) + z2 * jnp.float32(1 / 9))))
    lny = e.astype(jnp.float32) * jnp.float32(LN2) + 2.0 * z * p
    return -lny


def _sc_body(logits_hbm, targets_hbm, out_hbm,
             tb0, tb1, tb2, lb0, lb1, lb2,
             cti, ctv, ctl, cli, clv,
             tmem, lmem, mlv, obuf,
             st0, st1, st2, sl0, sl1, sl2):
    cid = lax.axis_index("c")
    sid = lax.axis_index("s")
    wid = sid * 2 + cid

    li = lax.iota(jnp.int32, 16)
    zi = jnp.zeros((16,), jnp.int32)
    tbufs = (tb0, tb1, tb2)
    lbufs = (lb0, lb1, lb2)
    tsems = (st0, st1, st2)
    lsems = (sl0, sl1, sl2)

    def collect_t(tch, lch, ivec, th, cur):
        m = tch >= th
        im = m.astype(jnp.int32)
        pos = cur + plsc.cumsum(im) - im
        posc = jnp.minimum(pos, CAP - 1)
        plsc.store_scatter(cti, [posc], ivec, mask=m)
        plsc.store_scatter(ctv, [posc], tch, mask=m)
        plsc.store_scatter(ctl, [posc], lch, mask=m)
        return cur + plsc.all_reduce_population_count(m)

    def collect_l(lch, ivec, th, cur):
        m = lch >= th
        im = m.astype(jnp.int32)
        pos = cur + plsc.cumsum(im) - im
        posc = jnp.minimum(pos, CAP - 1)
        plsc.store_scatter(cli, [posc], ivec, mask=m)
        plsc.store_scatter(clv, [posc], lch, mask=m)
        return cur + plsc.all_reduce_population_count(m)

    def scan_half(tb, lb, base, carry):
        if carry is None:
            t0 = tb[pl.ds(0, 16)]
            t1 = tb[pl.ds(16, 16)]
            x0 = lb[pl.ds(0, 16)]
            x1 = lb[pl.ds(16, 16)]
            curT = collect_t(t0, x0, li, NEG_BIG, zi)
            curT = collect_t(t1, x1, li + 16, NEG_BIG, curT)
            curL = collect_l(x0, li, NEG_BIG, zi)
            curL = collect_l(x1, li + 16, NEG_BIG, curL)
            At, Bt = t0, t1
            Al, Bl = x0, x1
            tht = jnp.min(jnp.minimum(At, Bt))
            thl = jnp.min(jnp.minimum(Al, Bl))
            for j in (2, 3):
                v = tb[pl.ds(16 * j, 16)]
                x = lb[pl.ds(16 * j, 16)]
                curT = collect_t(v, x, li + 16 * j, tht, curT)
                curL = collect_l(x, li + 16 * j, thl, curL)
                Bt = jnp.maximum(Bt, jnp.minimum(At, v))
                At = jnp.maximum(At, v)
                Bl = jnp.maximum(Bl, jnp.minimum(Al, x))
                Al = jnp.maximum(Al, x)
            tht = jnp.min(jnp.minimum(At, Bt))
            thl = jnp.min(jnp.minimum(Al, Bl))
            carry = (At, Bt, tht, Al, Bl, thl, curT, curL)
            gstart = 1
        else:
            gstart = 0

        def it(g, c):
            At, Bt, tht, Al, Bl, thl, curT, curL = c
            gbase = base + g * (16 * UNROLL)
            ts = [tb[pl.ds(g * 64 + 16 * j, 16)] for j in range(UNROLL)]
            xs = [lb[pl.ds(g * 64 + 16 * j, 16)] for j in range(UNROLL)]
            tmax = jnp.maximum(jnp.maximum(ts[0], ts[1]),
                               jnp.maximum(ts[2], ts[3]))
            xmax = jnp.maximum(jnp.maximum(xs[0], xs[1]),
                               jnp.maximum(xs[2], xs[3]))
            hit = jnp.any((tmax >= tht) | (xmax >= thl))
            nBt = jnp.maximum(Bt, jnp.minimum(At, tmax))
            nAt = jnp.maximum(At, tmax)
            nBl = jnp.maximum(Bl, jnp.minimum(Al, xmax))
            nAl = jnp.maximum(Al, xmax)

            def slow(cc):
                tht, thl, curT, curL = cc
                for j in range(UNROLL):
                    curT = collect_t(ts[j], xs[j], li + (gbase + 16 * j),
                                     tht, curT)
                for j in range(UNROLL):
                    curL = collect_l(xs[j], li + (gbase + 16 * j), thl, curL)
                return (jnp.min(jnp.minimum(nAt, nBt)),
                        jnp.min(jnp.minimum(nAl, nBl)),
                        curT, curL)

            tht, thl, curT, curL = lax.cond(
                hit, slow, lambda cc: cc, (tht, thl, curT, curL))
            return (nAt, nBt, tht, nAl, nBl, thl, curT, curL)

        return lax.fori_loop(gstart, NGROUP2, it, carry)

    def select30(cvals, cn):
        minf = jnp.float32(float("-inf"))
        st = _hl_init(cvals[pl.ds(0, 16)], cvals[pl.ds(16, 16)])
        nch = (cn + 15) // 16

        def it(i, st):
            vals = cvals[pl.ds(i * 16, 16)]
            valid = (li + i * 16) < cn
            return _merge_chunk(jnp.where(valid, vals, minf), st)

        return _hl_fin(lax.fori_loop(2, nch, it, st))

    def members_t(cn, tv, need):
        nch = (cn + 15) // 16

        def it(i, carry):
            tie, cur = carry
            vals = ctv[pl.ds(i * 16, 16)]
            idxv = cti[pl.ds(i * 16, 16)]
            lvals = ctl[pl.ds(i * 16, 16)]
            valid = (li + i * 16) < cn
            mg = valid & (vals > tv)
            me = valid & (vals == tv)
            ime = me.astype(jnp.int32)
            pe = plsc.cumsum(ime) - ime
            mm = mg | (me & (tie + pe < need))
            imm = mm.astype(jnp.int32)
            pos = cur + plsc.cumsum(imm) - imm
            plsc.store_scatter(tmem, [pos], idxv, mask=mm)
            plsc.store_scatter(mlv, [pos], lvals, mask=mm)
            return (tie + jnp.sum(ime),
                    cur + plsc.all_reduce_population_count(mm))

        lax.fori_loop(0, nch, it, (jnp.int32(0), zi))

    def members_l(cn, tv, need):
        nch = (cn + 15) // 16

        def it(i, carry):
            tie, cur = carry
            vals = clv[pl.ds(i * 16, 16)]
            idxv = cli[pl.ds(i * 16, 16)]
            valid = (li + i * 16) < cn
            mg = valid & (vals > tv)
            me = valid & (vals == tv)
            ime = me.astype(jnp.int32)
            pe = plsc.cumsum(ime) - ime
            mm = mg | (me & (tie + pe < need))
            imm = mm.astype(jnp.int32)
            pos = cur + plsc.cumsum(imm) - imm
            plsc.store_scatter(lmem, [pos], idxv, mask=mm)
            return (tie + jnp.sum(ime),
                    cur + plsc.all_reduce_population_count(mm))

        lax.fori_loop(0, nch, it, (jnp.int32(0), zi))

    def fb_scan_row(tbA, tbB, lbA, lbB):
        st_t = _hl_init(tbA[pl.ds(0, 16)], tbA[pl.ds(16, 16)])
        st_l = _hl_init(lbA[pl.ds(0, 16)], lbA[pl.ds(16, 16)])
        for j in (2, 3):
            st_t = _merge_chunk(tbA[pl.ds(16 * j, 16)], st_t)
            st_l = _merge_chunk(lbA[pl.ds(16 * j, 16)], st_l)

        def mk_it(tb, lb):
            def it(g, carry):
                st_t, st_l = carry
                ts = [tb[pl.ds(g * 64 + 16 * j, 16)] for j in range(UNROLL)]
                xs = [lb[pl.ds(g * 64 + 16 * j, 16)] for j in range(UNROLL)]
                tmax = jnp.maximum(jnp.maximum(ts[0], ts[1]),
                                   jnp.maximum(ts[2], ts[3]))
                xmax = jnp.maximum(jnp.maximum(xs[0], xs[1]),
                                   jnp.maximum(xs[2], xs[3]))
                hit = jnp.any((tmax > st_t[2]) | (xmax > st_l[2]))

                def slow(c):
                    st_t, st_l = c
                    for j in range(UNROLL):
                        st_t = _merge_chunk(ts[j], st_t)
                    for j in range(UNROLL):
                        st_l = _merge_chunk(xs[j], st_l)
                    return (st_t, st_l)

                return lax.cond(hit, slow, lambda c: c, carry)
            return it

        st_t, st_l = lax.fori_loop(1, NGROUP2, mk_it(tbA, lbA), (st_t, st_l))
        st_t, st_l = lax.fori_loop(0, NGROUP2, mk_it(tbB, lbB), (st_t, st_l))
        tvt, needt = _hl_fin(st_t)
        tvl, needl = _hl_fin(st_l)
        return tvt, needt, tvl, needl

    def fb_members_row(tbA, tbB, lbA, lbB, tvt, needt, tvl, needl):
        def chunk(t, x, iv, c):
            tieT, tieL, cur, ovv = c
            mTg = t > tvt
            mTe = t == tvt
            mLg = x > tvl
            mLe = x == tvl
            iTe = mTe.astype(jnp.int32)
            iLe = mLe.astype(jnp.int32)
            peT = plsc.cumsum(iTe) - iTe
            peL = plsc.cumsum(iLe) - iLe
            memT = mTg | (mTe & (tieT + peT < needt))
            memL = mLg | (mLe & (tieL + peL < needl))
            imT = memT.astype(jnp.int32)
            pos = cur + plsc.cumsum(imT) - imT
            plsc.store_scatter(tmem, [pos], iv, mask=memT)
            return (tieT + jnp.sum(iTe),
                    tieL + jnp.sum(iLe),
                    cur + plsc.all_reduce_population_count(memT),
                    ovv + (memT & memL).astype(jnp.int32))

        def mk_it(tb, lb, base):
            def it(g, carry):
                ts = [tb[pl.ds(g * 64 + 16 * j, 16)] for j in range(UNROLL)]
                xs = [lb[pl.ds(g * 64 + 16 * j, 16)] for j in range(UNROLL)]
                tmax = jnp.maximum(jnp.maximum(ts[0], ts[1]),
                                   jnp.maximum(ts[2], ts[3]))
                xmax = jnp.maximum(jnp.maximum(xs[0], xs[1]),
                                   jnp.maximum(xs[2], xs[3]))
                hit = jnp.any((tmax >= tvt) | (xmax >= tvl))

                def slow(c):
                    for j in range(UNROLL):
                        c = chunk(ts[j], xs[j],
                                  li + (base + g * 64 + 16 * j), c)
                    return c

                return lax.cond(hit, slow, lambda c: c, carry)
            return it

        zero = jnp.int32(0)
        c = lax.fori_loop(0, NGROUP2, mk_it(tbA, lbA, 0),
                          (zero, zero, zi, zi))
        c = lax.fori_loop(0, NGROUP2, mk_it(tbB, lbB, N2), c)
        t0 = tmem[pl.ds(0, 16)]
        t1 = tmem[pl.ds(16, 16)]

        def gath(idx):
            ia = jnp.minimum(jnp.maximum(idx, 0), N2 - 1)
            ib = jnp.minimum(jnp.maximum(idx - N2, 0), N2 - 1)
            ga = plsc.load_gather(lbA, [ia])
            gb = plsc.load_gather(lbB, [ib])
            return jnp.where(idx < N2, ga, gb)

        mlv[pl.ds(0, 16)] = gath(t0)
        mlv[pl.ds(16, 16)] = gath(t1)
        return jnp.sum(c[3])

    def phases(cnt, cnl, tbA, tbB, lbA, lbB):
        overflow = (cnt > CAP - 1) | (cnl > CAP - 1)

        def fast(_):
            tvt, needt = select30(ctv, cnt)
            tvl, needl = select30(clv, cnl)
            members_t(cnt, tvt, needt)
            members_l(cnl, tvl, needl)
            t0 = tmem[pl.ds(0, 16)]
            t1 = tmem[pl.ds(16, 16)]
            acc = jnp.zeros((16,), jnp.int32)
            for sh in range(16):
                perm = (li + sh) & 15
                r0 = plsc.load_gather(lmem, [perm])
                r1 = plsc.load_gather(lmem, [perm + 16])
                acc = (acc + (t0 == r0).astype(jnp.int32)
                       + (t0 == r1).astype(jnp.int32)
                       + (t1 == r0).astype(jnp.int32)
                       + (t1 == r1).astype(jnp.int32))
            return jnp.sum(acc)

        def slowfb(_):
            tvt, needt, tvl, needl = fb_scan_row(tbA, tbB, lbA, lbB)
            return fb_members_row(tbA, tbB, lbA, lbB,
                                  tvt, needt, tvl, needl)

        ov = lax.cond(overflow, slowfb, fast, None)
        g0 = mlv[pl.ds(0, 16)]
        g1 = mlv[pl.ds(16, 16)]
        f0 = _neg_log_sigmoid(g0)
        f1 = jnp.where(li < K - 16, _neg_log_sigmoid(g1), 0.0)
        fsum = jnp.sum(f0 + f1)
        w = 1.0 - ov.astype(jnp.float32) * jnp.float32(1.0 / K)
        return fsum * jnp.float32(1.0 / K) * w

    def issue(h):
        s = h % 3
        tcp = pltpu.async_copy(targets_hbm.at[wid * 8 + h], tbufs[s], tsems[s])
        lcp = pltpu.async_copy(logits_hbm.at[wid * 8 + h], lbufs[s], lsems[s])
        return (tcp, lcp)

    pend = [issue(0), issue(1)]
    lossvec = jnp.zeros((16,), jnp.float32)
    carry = None
    for h in range(8):
        s = h % 3
        tcp, lcp = pend[h]
        tcp.wait()
        lcp.wait()
        if h % 2 == 0:
            tmem[pl.ds(0, 16)] = jnp.full((16,), -1, jnp.int32)
            tmem[pl.ds(16, 16)] = jnp.full((16,), -1, jnp.int32)
            lmem[pl.ds(0, 16)] = jnp.full((16,), -2, jnp.int32)
            lmem[pl.ds(16, 16)] = jnp.full((16,), -2, jnp.int32)
            carry = scan_half(tbufs[s], lbufs[s], 0, None)
            if h + 2 < 8:
                pend.append(issue(h + 2))
        else:
            carry = scan_half(tbufs[s], lbufs[s], N2, carry)
            loss_r = phases(jnp.max(carry[6]), jnp.max(carry[7]),
                            tbufs[(h - 1) % 3], tbufs[s],
                            lbufs[(h - 1) % 3], lbufs[s])
            lossvec = jnp.where(li == h // 2, loss_r, lossvec)
            if h + 2 < 8:
                pend.append(issue(h + 2))

    obuf[...] = lossvec
    pltpu.sync_copy(obuf, out_hbm.at[wid])


@jax.jit
def _sc_call(logits, targets):
    fn = functools.partial(
        pl.kernel,
        out_type=jax.ShapeDtypeStruct((32, 16), jnp.float32),
        mesh=plsc.VectorSubcoreMesh(core_axis_name="c", subcore_axis_name="s"),
        compiler_params=pltpu.CompilerParams(needs_layout_passes=False),
        scratch_types=[
            pltpu.VMEM((N2,), jnp.float32),
            pltpu.VMEM((N2,), jnp.float32),
            pltpu.VMEM((N2,), jnp.float32),
            pltpu.VMEM((N2,), jnp.float32),
            pltpu.VMEM((N2,), jnp.float32),
            pltpu.VMEM((N2,), jnp.float32),
            pltpu.VMEM((CAP,), jnp.int32),
            pltpu.VMEM((CAP,), jnp.float32),
            pltpu.VMEM((CAP,), jnp.float32),
            pltpu.VMEM((CAP,), jnp.int32),
            pltpu.VMEM((CAP,), jnp.float32),
            pltpu.VMEM((32,), jnp.int32),
            pltpu.VMEM((32,), jnp.int32),
            pltpu.VMEM((32,), jnp.float32),
            pltpu.VMEM((16,), jnp.float32),
            pltpu.SemaphoreType.DMA,
            pltpu.SemaphoreType.DMA,
            pltpu.SemaphoreType.DMA,
            pltpu.SemaphoreType.DMA,
            pltpu.SemaphoreType.DMA,
            pltpu.SemaphoreType.DMA,
        ],
    )(_sc_body)
    part = fn(logits.reshape(B * 2, N2), targets.reshape(B * 2, N2))
    return jnp.sum(part) * jnp.float32(1.0 / B)


def kernel(logits, targets):
    return _sc_call(logits, targets)


# speedup vs baseline: 1.0003x; 1.0003x over previous
import functools

import jax
import jax.numpy as jnp
from jax import lax
from jax.experimental import pallas as pl
from jax.experimental.pallas import tpu as pltpu
from jax.experimental.pallas import tpu_sc as plsc

B = 128
N = 32768
N2 = N // 2
K = 30
UNROLL = 4
NGROUP2 = N2 // (16 * UNROLL)
CAP = 4096
EPS = 1e-07
LN2 = 0.6931471805599453
NEG_BIG = -3.4e38


def _sort_asc(v):
    return plsc.sort_key_val(v, v)[0]


def _sort_desc(v):
    return plsc.sort_key_val(v, v, descending=True)[0]


def _lane(vec, i):
    li = lax.iota(jnp.int32, 16)
    return jnp.max(jnp.where(li == i, vec, NEG_BIG))


def _merge_chunk(v, state):
    H, L, th = state
    hit = jnp.any(v > th)

    def merge(c):
        H, L, _ = c
        vs = _sort_asc(v)
        up = jnp.maximum(vs, L)
        upd = _sort_desc(up)
        nH = _sort_asc(jnp.maximum(H, upd))
        nL = _sort_desc(jnp.minimum(H, upd))
        return (nH, nL, jnp.min(nL))

    return lax.cond(hit, merge, lambda c: c, (H, L, th))


def _hl_init(c0, c1):
    h0 = _sort_asc(c0)
    l0 = _sort_desc(c1)
    H = _sort_asc(jnp.maximum(h0, l0))
    L = _sort_desc(jnp.minimum(h0, l0))
    return (H, L, jnp.min(L))


def _hl_fin(st):
    H, L, _ = st
    tv = _lane(L, K - 1 - 16)
    cgt = (jnp.sum((H > tv).astype(jnp.int32))
           + jnp.sum((L > tv).astype(jnp.int32)))
    return tv, K - cgt


def _neg_log_sigmoid(x):
    s = 1.0 / (1.0 + jnp.exp(-x))
    y = s + jnp.float32(EPS)
    bits = plsc.bitcast(y, jnp.int32)
    e = (bits >> 23) - 127
    m = plsc.bitcast((bits & 0x7FFFFF) | 0x3F800000, jnp.float32)
    z = (m - 1.0) / (m + 1.0)
    z2 = z * z
    p = 1.0 + z2 * (jnp.float32(1 / 3) + z2 * (jnp.float32(1 / 5)
          + z2 * (jnp.float32(1 / 7) + z2 * jnp.float32(1 / 9))))
    lny = e.astype(jnp.float32) * jnp.float32(LN2) + 2.0 * z * p
    return -lny


def _sc_body(logits_hbm, targets_hbm, out_hbm,
             tb0, tb1, tb2, lb0, lb1, lb2,
             cti, cli,
             tmem, lmem, mlv, obuf,
             st0, st1, st2, sl0, sl1, sl2):
    cid = lax.axis_index("c")
    sid = lax.axis_index("s")
    wid = sid * 2 + cid

    li = lax.iota(jnp.int32, 16)
    zi = jnp.zeros((16,), jnp.int32)
    tbufs = (tb0, tb1, tb2)
    lbufs = (lb0, lb1, lb2)
    tsems = (st0, st1, st2)
    lsems = (sl0, sl1, sl2)

    def collect_t(tch, lch, ivec, th, cur):
        m = tch >= th
        im = m.astype(jnp.int32)
        pos = cur + plsc.cumsum(im) - im
        posc = jnp.minimum(pos, CAP - 1)
        plsc.store_scatter(cti, [posc], ivec, mask=m)
        return cur + plsc.all_reduce_population_count(m)

    def collect_l(lch, ivec, th, cur):
        m = lch >= th
        im = m.astype(jnp.int32)
        pos = cur + plsc.cumsum(im) - im
        posc = jnp.minimum(pos, CAP - 1)
        plsc.store_scatter(cli, [posc], ivec, mask=m)
        return cur + plsc.all_reduce_population_count(m)

    def scan_half(tb, lb, base, carry):
        if carry is None:
            t0 = tb[pl.ds(0, 16)]
            t1 = tb[pl.ds(16, 16)]
            x0 = lb[pl.ds(0, 16)]
            x1 = lb[pl.ds(16, 16)]
            curT = collect_t(t0, x0, li, NEG_BIG, zi)
            curT = collect_t(t1, x1, li + 16, NEG_BIG, curT)
            curL = collect_l(x0, li, NEG_BIG, zi)
            curL = collect_l(x1, li + 16, NEG_BIG, curL)
            At, Bt = t0, t1
            Al, Bl = x0, x1
            tht = jnp.min(jnp.minimum(At, Bt))
            thl = jnp.min(jnp.minimum(Al, Bl))
            for j in (2, 3):
                v = tb[pl.ds(16 * j, 16)]
                x = lb[pl.ds(16 * j, 16)]
                curT = collect_t(v, x, li + 16 * j, tht, curT)
                curL = collect_l(x, li + 16 * j, thl, curL)
                Bt = jnp.maximum(Bt, jnp.minimum(At, v))
                At = jnp.maximum(At, v)
                Bl = jnp.maximum(Bl, jnp.minimum(Al, x))
                Al = jnp.maximum(Al, x)
            tht = jnp.min(jnp.minimum(At, Bt))
            thl = jnp.min(jnp.minimum(Al, Bl))
            carry = (At, Bt, tht, Al, Bl, thl, curT, curL)
            gstart = 1
        else:
            gstart = 0

        def it(g, c):
            At, Bt, tht, Al, Bl, thl, curT, curL = c
            gbase = base + g * (16 * UNROLL)
            ts = [tb[pl.ds(g * 64 + 16 * j, 16)] for j in range(UNROLL)]
            xs = [lb[pl.ds(g * 64 + 16 * j, 16)] for j in range(UNROLL)]
            tmax = jnp.maximum(jnp.maximum(ts[0], ts[1]),
                               jnp.maximum(ts[2], ts[3]))
            xmax = jnp.maximum(jnp.maximum(xs[0], xs[1]),
                               jnp.maximum(xs[2], xs[3]))
            hit = jnp.any((tmax >= tht) | (xmax >= thl))

            def slow(cc):
                At, Bt, tht, Al, Bl, thl, curT, curL = cc
                nBt = jnp.maximum(Bt, jnp.minimum(At, tmax))
                nAt = jnp.maximum(At, tmax)
                nBl = jnp.maximum(Bl, jnp.minimum(Al, xmax))
                nAl = jnp.maximum(Al, xmax)
                for j in range(UNROLL):
                    curT = collect_t(ts[j], xs[j], li + (gbase + 16 * j),
                                     tht, curT)
                for j in range(UNROLL):
                    curL = collect_l(xs[j], li + (gbase + 16 * j), thl, curL)
                return (nAt, nBt, jnp.min(jnp.minimum(nAt, nBt)),
                        nAl, nBl, jnp.min(jnp.minimum(nAl, nBl)),
                        curT, curL)

            return lax.cond(hit, slow, lambda cc: cc, c)

        return lax.fori_loop(gstart, NGROUP2, it, carry)

    def gather_split(idx, bufA, bufB):
        ia = jnp.minimum(jnp.maximum(idx, 0), N2 - 1)
        ib = jnp.minimum(jnp.maximum(idx - N2, 0), N2 - 1)
        ga = plsc.load_gather(bufA, [ia])
        gb = plsc.load_gather(bufB, [ib])
        return jnp.where(idx < N2, ga, gb)

    def select30(cref, cn, bufA, bufB):
        minf = jnp.float32(float("-inf"))

        def gv(i):
            return gather_split(cref[pl.ds(i * 16, 16)], bufA, bufB)

        st = _hl_init(gv(0), gv(1))
        nch = (cn + 15) // 16

        def it(i, st):
            valid = (li + i * 16) < cn
            return _merge_chunk(jnp.where(valid, gv(i), minf), st)

        return _hl_fin(lax.fori_loop(2, nch, it, st))

    def members_t(cn, tv, need, tbA, tbB, lbA, lbB):
        nch = (cn + 15) // 16

        def it(i, carry):
            tie, cur = carry
            idxv = cti[pl.ds(i * 16, 16)]
            vals = gather_split(idxv, tbA, tbB)
            lvals = gather_split(idxv, lbA, lbB)
            valid = (li + i * 16) < cn
            mg = valid & (vals > tv)
            me = valid & (vals == tv)
            ime = me.astype(jnp.int32)
            pe = plsc.cumsum(ime) - ime
            mm = mg | (me & (tie + pe < need))
            imm = mm.astype(jnp.int32)
            pos = cur + plsc.cumsum(imm) - imm
            plsc.store_scatter(tmem, [pos], idxv, mask=mm)
            plsc.store_scatter(mlv, [pos], lvals, mask=mm)
            return (tie + jnp.sum(ime),
                    cur + plsc.all_reduce_population_count(mm))

        lax.fori_loop(0, nch, it, (jnp.int32(0), zi))

    def members_l(cn, tv, need, lbA, lbB):
        nch = (cn + 15) // 16

        def it(i, carry):
            tie, cur = carry
            idxv = cli[pl.ds(i * 16, 16)]
            vals = gather_split(idxv, lbA, lbB)
            valid = (li + i * 16) < cn
            mg = valid & (vals > tv)
            me = valid & (vals == tv)
            ime = me.astype(jnp.int32)
            pe = plsc.cumsum(ime) - ime
            mm = mg | (me & (tie + pe < need))
            imm = mm.astype(jnp.int32)
            pos = cur + plsc.cumsum(imm) - imm
            plsc.store_scatter(lmem, [pos], idxv, mask=mm)
            return (tie + jnp.sum(ime),
                    cur + plsc.all_reduce_population_count(mm))

        lax.fori_loop(0, nch, it, (jnp.int32(0), zi))

    def fb_scan_row(tbA, tbB, lbA, lbB):
        st_t = _hl_init(tbA[pl.ds(0, 16)], tbA[pl.ds(16, 16)])
        st_l = _hl_init(lbA[pl.ds(0, 16)], lbA[pl.ds(16, 16)])
        for j in (2, 3):
            st_t = _merge_chunk(tbA[pl.ds(16 * j, 16)], st_t)
            st_l = _merge_chunk(lbA[pl.ds(16 * j, 16)], st_l)

        def mk_it(tb, lb):
            def it(g, carry):
                st_t, st_l = carry
                ts = [tb[pl.ds(g * 64 + 16 * j, 16)] for j in range(UNROLL)]
                xs = [lb[pl.ds(g * 64 + 16 * j, 16)] for j in range(UNROLL)]
                tmax = jnp.maximum(jnp.maximum(ts[0], ts[1]),
                                   jnp.maximum(ts[2], ts[3]))
                xmax = jnp.maximum(jnp.maximum(xs[0], xs[1]),
                                   jnp.maximum(xs[2], xs[3]))
                hit = jnp.any((tmax > st_t[2]) | (xmax > st_l[2]))

                def slow(c):
                    st_t, st_l = c
                    for j in range(UNROLL):
                        st_t = _merge_chunk(ts[j], st_t)
                    for j in range(UNROLL):
                        st_l = _merge_chunk(xs[j], st_l)
                    return (st_t, st_l)

                return lax.cond(hit, slow, lambda c: c, carry)
            return it

        st_t, st_l = lax.fori_loop(1, NGROUP2, mk_it(tbA, lbA), (st_t, st_l))
        st_t, st_l = lax.fori_loop(0, NGROUP2, mk_it(tbB, lbB), (st_t, st_l))
        tvt, needt = _hl_fin(st_t)
        tvl, needl = _hl_fin(st_l)
        return tvt, needt, tvl, needl

    def fb_members_row(tbA, tbB, lbA, lbB, tvt, needt, tvl, needl):
        def chunk(t, x, iv, c):
            tieT, tieL, cur, ovv = c
            mTg = t > tvt
            mTe = t == tvt
            mLg = x > tvl
            mLe = x == tvl
            iTe = mTe.astype(jnp.int32)
            iLe = mLe.astype(jnp.int32)
            peT = plsc.cumsum(iTe) - iTe
            peL = plsc.cumsum(iLe) - iLe
            memT = mTg | (mTe & (tieT + peT < needt))
            memL = mLg | (mLe & (tieL + peL < needl))
            imT = memT.astype(jnp.int32)
            pos = cur + plsc.cumsum(imT) - imT
            plsc.store_scatter(tmem, [pos], iv, mask=memT)
            return (tieT + jnp.sum(iTe),
                    tieL + jnp.sum(iLe),
                    cur + plsc.all_reduce_population_count(memT),
                    ovv + (memT & memL).astype(jnp.int32))

        def mk_it(tb, lb, base):
            def it(g, carry):
                ts = [tb[pl.ds(g * 64 + 16 * j, 16)] for j in range(UNROLL)]
                xs = [lb[pl.ds(g * 64 + 16 * j, 16)] for j in range(UNROLL)]
                tmax = jnp.maximum(jnp.maximum(ts[0], ts[1]),
                                   jnp.maximum(ts[2], ts[3]))
                xmax = jnp.maximum(jnp.maximum(xs[0], xs[1]),
                                   jnp.maximum(xs[2], xs[3]))
                hit = jnp.any((tmax >= tvt) | (xmax >= tvl))

                def slow(c):
                    for j in range(UNROLL):
                        c = chunk(ts[j], xs[j],
                                  li + (base + g * 64 + 16 * j), c)
                    return c

                return lax.cond(hit, slow, lambda c: c, carry)
            return it

        zero = jnp.int32(0)
        c = lax.fori_loop(0, NGROUP2, mk_it(tbA, lbA, 0),
                          (zero, zero, zi, zi))
        c = lax.fori_loop(0, NGROUP2, mk_it(tbB, lbB, N2), c)
        t0 = tmem[pl.ds(0, 16)]
        t1 = tmem[pl.ds(16, 16)]
        mlv[pl.ds(0, 16)] = gather_split(t0, lbA, lbB)
        mlv[pl.ds(16, 16)] = gather_split(t1, lbA, lbB)
        return jnp.sum(c[3])

    def phases(cnt, cnl, tbA, tbB, lbA, lbB):
        overflow = (cnt > CAP - 1) | (cnl > CAP - 1)

        def fast(_):
            tvt, needt = select30(cti, cnt, tbA, tbB)
            tvl, needl = select30(cli, cnl, lbA, lbB)
            members_t(cnt, tvt, needt, tbA, tbB, lbA, lbB)
            members_l(cnl, tvl, needl, lbA, lbB)
            t0 = tmem[pl.ds(0, 16)]
            t1 = tmem[pl.ds(16, 16)]
            acc = jnp.zeros((16,), jnp.int32)
            for sh in range(16):
                perm = (li + sh) & 15
                r0 = plsc.load_gather(lmem, [perm])
                r1 = plsc.load_gather(lmem, [perm + 16])
                acc = (acc + (t0 == r0).astype(jnp.int32)
                       + (t0 == r1).astype(jnp.int32)
                       + (t1 == r0).astype(jnp.int32)
                       + (t1 == r1).astype(jnp.int32))
            return jnp.sum(acc)

        def slowfb(_):
            tvt, needt, tvl, needl = fb_scan_row(tbA, tbB, lbA, lbB)
            return fb_members_row(tbA, tbB, lbA, lbB,
                                  tvt, needt, tvl, needl)

        ov = lax.cond(overflow, slowfb, fast, None)
        g0 = mlv[pl.ds(0, 16)]
        g1 = mlv[pl.ds(16, 16)]
        f0 = _neg_log_sigmoid(g0)
        f1 = jnp.where(li < K - 16, _neg_log_sigmoid(g1), 0.0)
        fsum = jnp.sum(f0 + f1)
        w = 1.0 - ov.astype(jnp.float32) * jnp.float32(1.0 / K)
        return fsum * jnp.float32(1.0 / K) * w

    def issue(h):
        s = h % 3
        tcp = pltpu.async_copy(targets_hbm.at[wid * 8 + h], tbufs[s], tsems[s])
        lcp = pltpu.async_copy(logits_hbm.at[wid * 8 + h], lbufs[s], lsems[s])
        return (tcp, lcp)

    pend = [issue(0), issue(1)]
    lossvec = jnp.zeros((16,), jnp.float32)
    carry = None
    tmem[pl.ds(0, 16)] = jnp.full((16,), -1, jnp.int32)
    tmem[pl.ds(16, 16)] = jnp.full((16,), -1, jnp.int32)
    lmem[pl.ds(0, 16)] = jnp.full((16,), -2, jnp.int32)
    lmem[pl.ds(16, 16)] = jnp.full((16,), -2, jnp.int32)
    for h in range(8):
        s = h % 3
        tcp, lcp = pend[h]
        tcp.wait()
        lcp.wait()
        if h % 2 == 0:
            carry = scan_half(tbufs[s], lbufs[s], 0, None)
            if h + 2 < 8:
                pend.append(issue(h + 2))
        else:
            carry = scan_half(tbufs[s], lbufs[s], N2, carry)
            loss_r = phases(jnp.max(carry[6]), jnp.max(carry[7]),
                            tbufs[(h - 1) % 3], tbufs[s],
                            lbufs[(h - 1) % 3], lbufs[s])
            lossvec = jnp.where(li == h // 2, loss_r, lossvec)
            if h + 2 < 8:
                pend.append(issue(h + 2))

    obuf[...] = lossvec
    pltpu.sync_copy(obuf, out_hbm.at[wid])


@jax.jit
def _sc_call(logits, targets):
    fn = functools.partial(
        pl.kernel,
        out_type=jax.ShapeDtypeStruct((32, 16), jnp.float32),
        mesh=plsc.VectorSubcoreMesh(core_axis_name="c", subcore_axis_name="s"),
        compiler_params=pltpu.CompilerParams(needs_layout_passes=False),
        scratch_types=[
            pltpu.VMEM((N2,), jnp.float32),
            pltpu.VMEM((N2,), jnp.float32),
            pltpu.VMEM((N2,), jnp.float32),
            pltpu.VMEM((N2,), jnp.float32),
            pltpu.VMEM((N2,), jnp.float32),
            pltpu.VMEM((N2,), jnp.float32),
            pltpu.VMEM((CAP,), jnp.int32),
            pltpu.VMEM((CAP,), jnp.int32),
            pltpu.VMEM((32,), jnp.int32),
            pltpu.VMEM((32,), jnp.int32),
            pltpu.VMEM((32,), jnp.float32),
            pltpu.VMEM((16,), jnp.float32),
            pltpu.SemaphoreType.DMA,
            pltpu.SemaphoreType.DMA,
            pltpu.SemaphoreType.DMA,
            pltpu.SemaphoreType.DMA,
            pltpu.SemaphoreType.DMA,
            pltpu.SemaphoreType.DMA,
        ],
    )(_sc_body)
    part = fn(logits.reshape(B * 2, N2), targets.reshape(B * 2, N2))
    return jnp.sum(part) * jnp.float32(1.0 / B)


def kernel(logits, targets):
    return _sc_call(logits, targets)


# speedup vs baseline: 1.2714x; 1.2710x over previous
import functools

import jax
import jax.numpy as jnp
from jax import lax
from jax.experimental import pallas as pl
from jax.experimental.pallas import tpu as pltpu
from jax.experimental.pallas import tpu_sc as plsc

B = 128
N = 32768
K = 30
NCHUNK = N // 16
UNROLL = 4
NGROUP = NCHUNK // UNROLL
CAP = 4096
EPS = 1e-07
LN2 = 0.6931471805599453
NEG_BIG = -3.4e38


def _any(m):
    return plsc.all_reduce_population_count(m)[0] > 0


def _sort_asc(v):
    return plsc.sort_key_val(v, v)[0]


def _sort_desc(v):
    return plsc.sort_key_val(v, v, descending=True)[0]


def _lane(vec, i):
    li = lax.iota(jnp.int32, 16)
    return jnp.max(jnp.where(li == i, vec, NEG_BIG))


def _merge_chunk(v, state):
    H, L, th = state
    hit = _any(v > th)

    def merge(c):
        H, L, _ = c
        vs = _sort_asc(v)
        up = jnp.maximum(vs, L)
        upd = _sort_desc(up)
        nH = _sort_asc(jnp.maximum(H, upd))
        nL = _sort_desc(jnp.minimum(H, upd))
        return (nH, nL, jnp.min(nL))

    return lax.cond(hit, merge, lambda c: c, (H, L, th))


def _hl_init(c0, c1):
    h0 = _sort_asc(c0)
    l0 = _sort_desc(c1)
    H = _sort_asc(jnp.maximum(h0, l0))
    L = _sort_desc(jnp.minimum(h0, l0))
    return (H, L, jnp.min(L))


def _hl_fin(st):
    H, L, _ = st
    tv = _lane(L, K - 1 - 16)
    cgt = (jnp.sum((H > tv).astype(jnp.int32))
           + jnp.sum((L > tv).astype(jnp.int32)))
    return tv, K - cgt


def _collect(v, ivec, th, cur, cref):
    m = v >= th
    im = m.astype(jnp.int32)
    pos = cur + plsc.cumsum(im) - im
    posc = jnp.minimum(pos, CAP - 1)
    plsc.store_scatter(cref, [posc], ivec, mask=m)
    return cur + plsc.all_reduce_population_count(m)


def _scan_collect2(tb, lb, candT, candL):
    li = lax.iota(jnp.int32, 16)
    zi = jnp.zeros((16,), jnp.int32)

    t0 = tb[pl.ds(0, 16)]
    t1 = tb[pl.ds(16, 16)]
    x0 = lb[pl.ds(0, 16)]
    x1 = lb[pl.ds(16, 16)]
    curT = _collect(t0, li, NEG_BIG, zi, candT)
    curT = _collect(t1, li + 16, NEG_BIG, curT, candT)
    curL = _collect(x0, li, NEG_BIG, zi, candL)
    curL = _collect(x1, li + 16, NEG_BIG, curL, candL)
    At, Bt = t0, t1
    Al, Bl = x0, x1
    tht = jnp.min(jnp.minimum(At, Bt))
    thl = jnp.min(jnp.minimum(Al, Bl))
    for j in (2, 3):
        v = tb[pl.ds(16 * j, 16)]
        curT = _collect(v, li + 16 * j, tht, curT, candT)
        Bt = jnp.maximum(Bt, jnp.minimum(At, v))
        At = jnp.maximum(At, v)
        x = lb[pl.ds(16 * j, 16)]
        curL = _collect(x, li + 16 * j, thl, curL, candL)
        Bl = jnp.maximum(Bl, jnp.minimum(Al, x))
        Al = jnp.maximum(Al, x)
    tht = jnp.min(jnp.minimum(At, Bt))
    thl = jnp.min(jnp.minimum(Al, Bl))

    def it(g, carry):
        At, Bt, tht, Al, Bl, thl, curT, curL = carry
        base = g * (16 * UNROLL)
        ts = [tb[pl.ds(base + 16 * j, 16)] for j in range(UNROLL)]
        xs = [lb[pl.ds(base + 16 * j, 16)] for j in range(UNROLL)]
        tmax = jnp.maximum(jnp.maximum(ts[0], ts[1]), jnp.maximum(ts[2], ts[3]))
        xmax = jnp.maximum(jnp.maximum(xs[0], xs[1]), jnp.maximum(xs[2], xs[3]))
        hit = _any((tmax >= tht) | (xmax >= thl))

        def slow(c):
            At, Bt, tht, Al, Bl, thl, curT, curL = c
            nBt = jnp.maximum(Bt, jnp.minimum(At, tmax))
            nAt = jnp.maximum(At, tmax)
            nBl = jnp.maximum(Bl, jnp.minimum(Al, xmax))
            nAl = jnp.maximum(Al, xmax)
            for j in range(UNROLL):
                curT = _collect(ts[j], li + (base + 16 * j), tht, curT, candT)
            for j in range(UNROLL):
                curL = _collect(xs[j], li + (base + 16 * j), thl, curL, candL)
            return (nAt, nBt, jnp.min(jnp.minimum(nAt, nBt)),
                    nAl, nBl, jnp.min(jnp.minimum(nAl, nBl)),
                    curT, curL)

        return lax.cond(hit, slow, lambda c: c, carry)

    carry = lax.fori_loop(1, NGROUP, it,
                          (At, Bt, tht, Al, Bl, thl, curT, curL))
    return jnp.max(carry[6]), jnp.max(carry[7])


def _select30(cref, cn, buf):
    li = lax.iota(jnp.int32, 16)
    minf = jnp.float32(float("-inf"))

    def gather(i):
        idxv = cref[pl.ds(i * 16, 16)]
        idxg = jnp.minimum(jnp.maximum(idxv, 0), N - 1)
        return plsc.load_gather(buf, [idxg])

    st = _hl_init(gather(0), gather(1))
    nch = (cn + 15) // 16

    def it(i, st):
        vals = gather(i)
        valid = (li + i * 16) < cn
        v = jnp.where(valid, vals, minf)
        return _merge_chunk(v, st)

    st = lax.fori_loop(2, nch, it, st)
    return _hl_fin(st)


def _scan_topk2_full(tb, lb):
    st_t = _hl_init(tb[pl.ds(0, 16)], tb[pl.ds(16, 16)])
    st_l = _hl_init(lb[pl.ds(0, 16)], lb[pl.ds(16, 16)])
    for j in (2, 3):
        st_t = _merge_chunk(tb[pl.ds(16 * j, 16)], st_t)
        st_l = _merge_chunk(lb[pl.ds(16 * j, 16)], st_l)

    def it(g, carry):
        st_t, st_l = carry
        base = g * (16 * UNROLL)
        ts = [tb[pl.ds(base + 16 * j, 16)] for j in range(UNROLL)]
        xs = [lb[pl.ds(base + 16 * j, 16)] for j in range(UNROLL)]
        tmax = jnp.maximum(jnp.maximum(ts[0], ts[1]), jnp.maximum(ts[2], ts[3]))
        xmax = jnp.maximum(jnp.maximum(xs[0], xs[1]), jnp.maximum(xs[2], xs[3]))
        hit = _any((tmax > st_t[2]) | (xmax > st_l[2]))

        def slow(c):
            st_t, st_l = c
            for j in range(UNROLL):
                st_t = _merge_chunk(ts[j], st_t)
            for j in range(UNROLL):
                st_l = _merge_chunk(xs[j], st_l)
            return (st_t, st_l)

        return lax.cond(hit, slow, lambda c: c, carry)

    st_t, st_l = lax.fori_loop(1, NGROUP, it, (st_t, st_l))
    tvt, needt = _hl_fin(st_t)
    tvl, needl = _hl_fin(st_l)
    return tvt, needt, tvl, needl


def _cand_members(cref, cn, buf, tv, need, mref):
    li = lax.iota(jnp.int32, 16)
    zi = jnp.zeros((16,), jnp.int32)
    nch = (cn + 15) // 16

    def it(i, carry):
        tie, cur = carry
        idxv = cref[pl.ds(i * 16, 16)]
        idxg = jnp.minimum(jnp.maximum(idxv, 0), N - 1)
        vals = plsc.load_gather(buf, [idxg])
        valid = (li + i * 16) < cn
        mg = valid & (vals > tv)
        me = valid & (vals == tv)
        ime = me.astype(jnp.int32)
        pe = plsc.cumsum(ime) - ime
        mm = mg | (me & (tie + pe < need))
        imm = mm.astype(jnp.int32)
        pos = cur + plsc.cumsum(imm) - imm
        plsc.store_scatter(mref, [pos], idxv, mask=mm)
        return (tie + jnp.sum(ime), cur + plsc.all_reduce_population_count(mm))

    lax.fori_loop(0, nch, it, (jnp.int32(0), zi))


def _membership_full(tb, lb, tvt, needt, tvl, needl, tmem):
    li = lax.iota(jnp.int32, 16)
    zero = jnp.int32(0)
    zi = jnp.zeros((16,), jnp.int32)

    def chunk(t, x, iv, c):
        tieT, tieL, cur, ovv = c
        mTg = t > tvt
        mTe = t == tvt
        mLg = x > tvl
        mLe = x == tvl
        iTe = mTe.astype(jnp.int32)
        iLe = mLe.astype(jnp.int32)
        peT = plsc.cumsum(iTe) - iTe
        peL = plsc.cumsum(iLe) - iLe
        memT = mTg | (mTe & (tieT + peT < needt))
        memL = mLg | (mLe & (tieL + peL < needl))
        imT = memT.astype(jnp.int32)
        pos = cur + plsc.cumsum(imT) - imT
        plsc.store_scatter(tmem, [pos], iv, mask=memT)
        return (tieT + jnp.sum(iTe),
                tieL + jnp.sum(iLe),
                cur + plsc.all_reduce_population_count(memT),
                ovv + (memT & memL).astype(jnp.int32))

    def it(g, carry):
        base = g * (16 * UNROLL)
        ts = [tb[pl.ds(base + 16 * j, 16)] for j in range(UNROLL)]
        xs = [lb[pl.ds(base + 16 * j, 16)] for j in range(UNROLL)]
        tmax = jnp.maximum(jnp.maximum(ts[0], ts[1]), jnp.maximum(ts[2], ts[3]))
        xmax = jnp.maximum(jnp.maximum(xs[0], xs[1]), jnp.maximum(xs[2], xs[3]))
        hit = _any((tmax >= tvt) | (xmax >= tvl))

        def slow(c):
            for j in range(UNROLL):
                c = chunk(ts[j], xs[j], li + (base + 16 * j), c)
            return c

        return lax.cond(hit, slow, lambda c: c, carry)

    carry = lax.fori_loop(0, NGROUP, it, (zero, zero, zi, zi))
    return jnp.sum(carry[3])


def _neg_log_sigmoid(x):
    s = 1.0 / (1.0 + jnp.exp(-x))
    y = s + jnp.float32(EPS)
    bits = plsc.bitcast(y, jnp.int32)
    e = (bits >> 23) - 127
    m = plsc.bitcast((bits & 0x7FFFFF) | 0x3F800000, jnp.float32)
    z = (m - 1.0) / (m + 1.0)
    z2 = z * z
    p = 1.0 + z2 * (jnp.float32(1 / 3) + z2 * (jnp.float32(1 / 5)
          + z2 * (jnp.float32(1 / 7) + z2 * jnp.float32(1 / 9))))
    lny = e.astype(jnp.float32) * jnp.float32(LN2) + 2.0 * z * p
    return -lny


def _sc_body(logits_hbm, targets_hbm, out_hbm,
             tbuf, lbuf, candT, candL, tmem, lmem, obuf, semt, seml):
    cid = lax.axis_index("c")
    sid = lax.axis_index("s")
    wid = sid * 2 + cid

    li = lax.iota(jnp.int32, 16)

    def row_it(r, lossvec):
        row = wid * 4 + r
        ct = pltpu.async_copy(targets_hbm.at[row], tbuf, semt)
        cl = pltpu.async_copy(logits_hbm.at[row], lbuf, seml)
        ct.wait()
        cl.wait()
        tmem[pl.ds(0, 16)] = jnp.full((16,), -1, jnp.int32)
        tmem[pl.ds(16, 16)] = jnp.full((16,), -1, jnp.int32)
        lmem[pl.ds(0, 16)] = jnp.full((16,), -2, jnp.int32)
        lmem[pl.ds(16, 16)] = jnp.full((16,), -2, jnp.int32)

        cnt, cnl = _scan_collect2(tbuf, lbuf, candT, candL)
        overflow = (cnt > CAP - 1) | (cnl > CAP - 1)

        def fast(_):
            tvt, needt = _select30(candT, cnt, tbuf)
            tvl, needl = _select30(candL, cnl, lbuf)
            _cand_members(candT, cnt, tbuf, tvt, needt, tmem)
            _cand_members(candL, cnl, lbuf, tvl, needl, lmem)
            t0 = tmem[pl.ds(0, 16)]
            t1 = tmem[pl.ds(16, 16)]
            acc = jnp.zeros((16,), jnp.int32)
            for sh in range(16):
                perm = (li + sh) & 15
                r0 = plsc.load_gather(lmem, [perm])
                r1 = plsc.load_gather(lmem, [perm + 16])
                acc = (acc + (t0 == r0).astype(jnp.int32)
                       + (t0 == r1).astype(jnp.int32)
                       + (t1 == r0).astype(jnp.int32)
                       + (t1 == r1).astype(jnp.int32))
            return jnp.sum(acc)

        def slowfb(_):
            tvt, needt, tvl, needl = _scan_topk2_full(tbuf, lbuf)
            return _membership_full(tbuf, lbuf, tvt, needt, tvl, needl, tmem)

        ov = lax.cond(overflow, slowfb, fast, None)

        t0 = jnp.maximum(tmem[pl.ds(0, 16)], 0)
        t1 = jnp.maximum(tmem[pl.ds(16, 16)], 0)
        g0 = plsc.load_gather(lbuf, [t0])
        g1 = plsc.load_gather(lbuf, [t1])
        f0 = _neg_log_sigmoid(g0)
        f1 = jnp.where(li < K - 16, _neg_log_sigmoid(g1), 0.0)
        fsum = jnp.sum(f0 + f1)
        w = 1.0 - ov.astype(jnp.float32) * jnp.float32(1.0 / K)
        loss_r = fsum * jnp.float32(1.0 / K) * w
        return jnp.where(li == r, loss_r, lossvec)

    lossvec = lax.fori_loop(0, 4, row_it, jnp.zeros((16,), jnp.float32))
    obuf[...] = lossvec
    pltpu.sync_copy(obuf, out_hbm.at[wid])


@jax.jit
def _sc_call(logits, targets):
    fn = functools.partial(
        pl.kernel,
        out_type=jax.ShapeDtypeStruct((32, 16), jnp.float32),
        mesh=plsc.VectorSubcoreMesh(core_axis_name="c", subcore_axis_name="s"),
        compiler_params=pltpu.CompilerParams(needs_layout_passes=False),
        scratch_types=[
            pltpu.VMEM((N,), jnp.float32),
            pltpu.VMEM((N,), jnp.float32),
            pltpu.VMEM((CAP,), jnp.int32),
            pltpu.VMEM((CAP,), jnp.int32),
            pltpu.VMEM((32,), jnp.int32),
            pltpu.VMEM((32,), jnp.int32),
            pltpu.VMEM((16,), jnp.float32),
            pltpu.SemaphoreType.DMA,
            pltpu.SemaphoreType.DMA,
        ],
    )(_sc_body)
    part = fn(logits, targets)
    return jnp.sum(part) * jnp.float32(1.0 / B)


def kernel(logits, targets):
    return _sc_call(logits, targets)
